# baseline (speedup 1.0000x reference)
import functools

import jax
import jax.numpy as jnp
from jax import lax
from jax.experimental import pallas as pl
from jax.experimental.pallas import tpu as pltpu
from jax.experimental.pallas import tpu_sc as plsc

BATCH = 16384
SEQ = 50
EMB = 64
TOTAL = BATCH * SEQ

_INFO = plsc.get_sparse_core_info()
_NC = _INFO.num_cores
_NS = _INFO.num_subcores
_NW = _NC * _NS
_BPW = TOTAL // _NW
_CHUNK = 400
_NCHUNK = _BPW // _CHUNK
_NBUF = 4
_NGROUP = _NCHUNK // _NBUF
_NBG = BATCH // 128


def _make_sc_gather():
    mesh = plsc.VectorSubcoreMesh(core_axis_name="c", subcore_axis_name="s")

    @functools.partial(
        pl.kernel,
        mesh=mesh,
        out_type=jax.ShapeDtypeStruct((TOTAL, EMB), jnp.float32),
        compiler_params=pltpu.CompilerParams(use_tc_tiling_on_sc=False),
        scratch_types=[
            pltpu.VMEM((_BPW,), jnp.int32),
            pltpu.VMEM((_NBUF, _CHUNK, EMB), jnp.float32),
            pltpu.SemaphoreType.DMA((_NBUF,)),
            pltpu.SemaphoreType.DMA((_NBUF,)),
        ],
    )
    def gather_kernel(idx_hbm, table_hbm, out_hbm, idx_v, rows_v, gsem, ssem):
        wid = lax.axis_index("s") * _NC + lax.axis_index("c")
        base = wid * _BPW

        pltpu.sync_copy(idx_hbm.at[pl.ds(base, _BPW)], idx_v)

        def start_gather(g, b):
            pltpu.async_copy(
                table_hbm.at[idx_v.at[pl.ds(g * _CHUNK, _CHUNK)]],
                rows_v.at[b],
                gsem.at[b],
            )

        def wait_gather(b):
            pltpu.make_async_copy(
                table_hbm.at[pl.ds(0, _CHUNK)], rows_v.at[b], gsem.at[b]
            ).wait()

        def start_store(g, b):
            pltpu.async_copy(
                rows_v.at[b],
                out_hbm.at[pl.ds(base + g * _CHUNK, _CHUNK)],
                ssem.at[b],
            )

        def wait_store(b):
            pltpu.make_async_copy(
                rows_v.at[b], out_hbm.at[pl.ds(base, _CHUNK)], ssem.at[b]
            ).wait()

        for b in range(_NBUF):
            start_gather(b, b)

        def body(j, carry):
            g0 = j * _NBUF
            for b in range(_NBUF):
                wait_gather(b)
                start_store(g0 + b, b)
            for b in range(_NBUF):
                wait_store(b)
                start_gather(g0 + _NBUF + b, b)
            return carry

        lax.fori_loop(0, _NGROUP - 1, body, 0)

        g0 = (_NGROUP - 1) * _NBUF
        for b in range(_NBUF):
            wait_gather(b)
            start_store(g0 + b, b)
        for b in range(_NBUF):
            wait_store(b)

    return gather_kernel


_sc_gather = _make_sc_gather()


def _tc_relayout_body(in_ref, out_ref):
    x2 = in_ref[...]
    a = x2[:, :EMB]
    b = x2[:, EMB:]
    y = jnp.concatenate([a.T, b.T], axis=1)
    out_ref[...] = y.reshape(1, 8, 1, 8, 128)


_tc_relayout = pl.pallas_call(
    _tc_relayout_body,
    grid=(SEQ, _NBG),
    in_specs=[
        pl.BlockSpec((64, 128), lambda s, bg: (s * _NBG + bg, 0)),
    ],
    out_specs=pl.BlockSpec(
        (1, 8, 1, 8, 128), lambda s, bg: (s, 0, bg, 0, 0)
    ),
    out_shape=jax.ShapeDtypeStruct((SEQ, 8, _NBG, 8, 128), jnp.float32),
)


def kernel(x, table):
    xt = (
        jnp.transpose(x)
        .reshape(SEQ, _NBG, 2, EMB)
        .swapaxes(2, 3)
        .reshape(TOTAL)
        .astype(jnp.int32)
    )
    flat = _sc_gather(xt, table)
    ltiles = _tc_relayout(flat.reshape(TOTAL // 2, 2 * EMB))
    return jnp.transpose(ltiles, (2, 4, 0, 1, 3)).reshape(BATCH, SEQ, EMB)


# speedup vs baseline: 3.5794x; 3.5794x over previous
import functools

import jax
import jax.numpy as jnp
from jax import lax
from jax.experimental import pallas as pl
from jax.experimental.pallas import tpu as pltpu
from jax.experimental.pallas import tpu_sc as plsc

BATCH = 16384
SEQ = 50
EMB = 64
TOTAL = BATCH * SEQ

_INFO = plsc.get_sparse_core_info()
_NC = _INFO.num_cores
_NS = _INFO.num_subcores
_NW = _NC * _NS
_BPW = TOTAL // _NW
_CHUNK = 400
_NCHUNK = _BPW // _CHUNK
_NBUF = 4
_NGROUP = _NCHUNK // _NBUF


def _make_sc_gather():
    mesh = plsc.VectorSubcoreMesh(core_axis_name="c", subcore_axis_name="s")

    @functools.partial(
        pl.kernel,
        mesh=mesh,
        out_type=jax.ShapeDtypeStruct((TOTAL, EMB), jnp.float32),
        compiler_params=pltpu.CompilerParams(use_tc_tiling_on_sc=False),
        scratch_types=[
            pltpu.VMEM((_BPW,), jnp.int32),
            pltpu.VMEM((_NBUF, _CHUNK, EMB), jnp.float32),
            pltpu.SemaphoreType.DMA((_NBUF,)),
            pltpu.SemaphoreType.DMA((_NBUF,)),
        ],
    )
    def gather_kernel(idx_hbm, table_hbm, out_hbm, idx_v, rows_v, gsem, ssem):
        wid = lax.axis_index("s") * _NC + lax.axis_index("c")
        base = wid * _BPW

        pltpu.sync_copy(idx_hbm.at[pl.ds(base, _BPW)], idx_v)

        def start_gather(g, b):
            pltpu.async_copy(
                table_hbm.at[idx_v.at[pl.ds(g * _CHUNK, _CHUNK)]],
                rows_v.at[b],
                gsem.at[b],
            )

        def wait_gather(b):
            pltpu.make_async_copy(
                table_hbm.at[pl.ds(0, _CHUNK)], rows_v.at[b], gsem.at[b]
            ).wait()

        def start_store(g, b):
            pltpu.async_copy(
                rows_v.at[b],
                out_hbm.at[pl.ds(base + g * _CHUNK, _CHUNK)],
                ssem.at[b],
            )

        def wait_store(b):
            pltpu.make_async_copy(
                rows_v.at[b], out_hbm.at[pl.ds(base, _CHUNK)], ssem.at[b]
            ).wait()

        for b in range(_NBUF):
            start_gather(b, b)

        def body(j, carry):
            g0 = j * _NBUF
            for b in range(_NBUF):
                wait_gather(b)
                start_store(g0 + b, b)
            for b in range(_NBUF):
                wait_store(b)
                start_gather(g0 + _NBUF + b, b)
            return carry

        lax.fori_loop(0, _NGROUP - 1, body, 0)

        g0 = (_NGROUP - 1) * _NBUF
        for b in range(_NBUF):
            wait_gather(b)
            start_store(g0 + b, b)
        for b in range(_NBUF):
            wait_store(b)

    return gather_kernel


_sc_gather = _make_sc_gather()


def kernel(x, table):
    x_flat = x.reshape(TOTAL).astype(jnp.int32)
    out = _sc_gather(x_flat, table)
    return out.reshape(BATCH, SEQ, EMB)
